# Initial kernel scaffold; baseline (speedup 1.0000x reference)
import jax
import jax.numpy as jnp
from jax.experimental import pallas as pl


def kernel(data, emb):
    raise NotImplementedError("write your pallas kernel here")


# speedup vs baseline: 7.6528x; 7.6528x over previous
import functools

import jax
import jax.numpy as jnp
from jax import lax
from jax.experimental import pallas as pl
from jax.experimental.pallas import tpu as pltpu
from jax.experimental.pallas import tpu_sc as plsc

N_NODES = 10000
EMBED_DIM = 128
N_EDGES = 320000

NC = 2
NS = 16
NW = NC * NS
EPW = N_EDGES // NW
CHUNK = 80
NCHUNK = EPW // CHUNK
NGROUP = CHUNK // 16


def _body(emb_hbm, d0_hbm, d1_hbm, out_hbm,
          idx0_v, idx1_v, r0a, r0b, r1a, r1b, out_v, tbuf_v, sem0, sem1):
  wid = lax.axis_index("s") * NC + lax.axis_index("c")
  base = wid * EPW

  pltpu.sync_copy(d0_hbm.at[pl.ds(base, EPW)], idx0_v)
  pltpu.sync_copy(d1_hbm.at[pl.ds(base, EPW)], idx1_v)

  bufs = ((r0a, r1a, sem0), (r0b, r1b, sem1))

  def issue(c, b):
    r0, r1, sem = bufs[b]
    off = c * CHUNK
    pltpu.async_copy(emb_hbm.at[idx0_v.at[pl.ds(off, CHUNK)]], r0, sem)
    pltpu.async_copy(emb_hbm.at[idx1_v.at[pl.ds(off, CHUNK)]], r1, sem)

  def drain(c, b):
    r0, r1, sem = bufs[b]
    off = c * CHUNK
    pltpu.make_async_copy(emb_hbm.at[idx0_v.at[pl.ds(off, CHUNK)]], r0, sem).wait()
    pltpu.make_async_copy(emb_hbm.at[idx1_v.at[pl.ds(off, CHUNK)]], r1, sem).wait()

  lane = lax.iota(jnp.int32, 16)

  def compute(c, b):
    r0, r1, _ = bufs[b]

    def group_body(g, carry):
      for i in range(16):
        r = g * 16 + i
        s = r0[r, pl.ds(0, 16)] * r1[r, pl.ds(0, 16)]
        for j in range(1, EMBED_DIM // 16):
          s = s + r0[r, pl.ds(j * 16, 16)] * r1[r, pl.ds(j * 16, 16)]
        tbuf_v[pl.ds(i * 16, 16)] = s
      acc = plsc.load_gather(tbuf_v, [lane * 16])
      for l in range(1, 16):
        acc = acc + plsc.load_gather(tbuf_v, [lane * 16 + l])
      out_v[pl.ds(c * CHUNK + g * 16, 16)] = acc
      return carry

    lax.fori_loop(0, NGROUP, group_body, 0)

  issue(0, 0)
  issue(1, 1)

  def chunk_body(i, carry):
    for b in range(2):
      c = 2 * i + b

      @pl.when(c < NCHUNK)
      def _do():
        drain(c, b)
        compute(c, b)

        @pl.when(c + 2 < NCHUNK)
        def _next():
          issue(c + 2, b)

    return carry

  lax.fori_loop(0, (NCHUNK + 1) // 2, chunk_body, 0)

  pltpu.sync_copy(out_v, out_hbm.at[pl.ds(base, EPW)])


_sc_call = functools.partial(
    pl.kernel,
    out_type=jax.ShapeDtypeStruct((N_EDGES,), jnp.float32),
    mesh=plsc.VectorSubcoreMesh(core_axis_name="c", subcore_axis_name="s"),
    compiler_params=pltpu.CompilerParams(needs_layout_passes=False),
    scratch_types=[
        pltpu.VMEM((EPW,), jnp.int32),
        pltpu.VMEM((EPW,), jnp.int32),
        pltpu.VMEM((CHUNK, EMBED_DIM), jnp.float32),
        pltpu.VMEM((CHUNK, EMBED_DIM), jnp.float32),
        pltpu.VMEM((CHUNK, EMBED_DIM), jnp.float32),
        pltpu.VMEM((CHUNK, EMBED_DIM), jnp.float32),
        pltpu.VMEM((EPW,), jnp.float32),
        pltpu.VMEM((256,), jnp.float32),
        pltpu.SemaphoreType.DMA,
        pltpu.SemaphoreType.DMA,
    ],
)(_body)


@jax.jit
def kernel(data, emb):
  return _sc_call(emb, data[0], data[1])
